# baseline (speedup 1.0000x reference)
import functools

import jax
import jax.numpy as jnp
from jax import lax
from jax.experimental import pallas as pl
from jax.experimental.pallas import tpu as pltpu
from jax.experimental.pallas import tpu_sc as plsc

N_NODES = 10000
N_EDGES = 320000
D = 128

NC = 2
NS = 16
NW = NC * NS
CS = 128
CH = 80
EPW_PAD = CH * CS
ACC_ROWS = 10240
RPT = ACC_ROWS // NS
L = 16


def _sc_segment_sum(edge_idx, x):
  mesh = plsc.VectorSubcoreMesh(core_axis_name="c", subcore_axis_name="s")

  @functools.partial(
      pl.kernel,
      mesh=mesh,
      out_type=[
          jax.ShapeDtypeStruct((NC, ACC_ROWS, D), jnp.float32),
          jax.ShapeDtypeStruct((NC, NS, ACC_ROWS), jnp.float32),
      ],
      scratch_types=[
          pltpu.VMEM((2, CS), jnp.int32),
          pltpu.VMEM((2, CS), jnp.int32),
          pltpu.VMEM((CS, D), jnp.float32),
          pltpu.VMEM((CS, D), jnp.float32),
          pltpu.VMEM((ACC_ROWS,), jnp.float32),
          pltpu.VMEM_SHARED((ACC_ROWS, D), jnp.float32),
          pltpu.SemaphoreType.DMA,
          pltpu.SemaphoreType.DMA,
      ],
      compiler_params=pltpu.CompilerParams(needs_layout_passes=False),
  )
  def seg_kernel(idx_hbm, x_hbm, acc_out, deg_out,
                 idx_a, idx_b, rows_a, rows_b, deg_v, acc_sh, sem_a, sem_b):
    c = lax.axis_index("c")
    s = lax.axis_index("s")
    wid = c * NS + s
    idx_bufs = (idx_a, idx_b)
    row_bufs = (rows_a, rows_b)
    sems = (sem_a, sem_b)

    def fill_zrows(i, _):
      r = i // (D // L)
      col = (i % (D // L)) * L
      rows_a[r, pl.ds(col, L)] = jnp.zeros((L,), jnp.float32)
      return 0
    lax.fori_loop(0, CS * (D // L), fill_zrows, 0)

    def fill_deg(i, _):
      deg_v[pl.ds(i * L, L)] = jnp.zeros((L,), jnp.float32)
      return 0
    lax.fori_loop(0, ACC_ROWS // L, fill_deg, 0)

    base = s * RPT
    for k in range(RPT // CS):
      pltpu.sync_copy(rows_a, acc_sh.at[pl.ds(base + k * CS, CS)])
    plsc.subcore_barrier()

    pltpu.sync_copy(idx_hbm.at[wid, 0], idx_a)
    pltpu.async_copy(x_hbm.at[idx_a.at[0]], rows_a, sem_a)
    pltpu.sync_copy(idx_hbm.at[wid, 1], idx_b)

    ones_l = jnp.ones((L,), jnp.float32)

    def pair(i, _):
      for b in range(2):
        m = 2 * i + b
        nb = 1 - b

        @pl.when(m + 1 < CH)
        def _():
          pltpu.async_copy(x_hbm.at[idx_bufs[nb].at[0]], row_bufs[nb],
                           sems[nb])

        for g in range(CS // L):
          plsc.addupdate_scatter(
              deg_v, [idx_bufs[b][1, pl.ds(g * L, L)]], ones_l)

        pltpu.make_async_copy(x_hbm.at[idx_bufs[b].at[0]], row_bufs[b],
                              sems[b]).wait()
        pltpu.sync_copy(row_bufs[b], acc_sh.at[idx_bufs[b].at[1]], add=True)

        @pl.when(m + 2 < CH)
        def _():
          pltpu.sync_copy(idx_hbm.at[wid, m + 2], idx_bufs[b])
      return 0
    lax.fori_loop(0, CH // 2, pair, 0)

    pltpu.sync_copy(deg_v, deg_out.at[c, s])
    plsc.subcore_barrier()
    pltpu.sync_copy(acc_sh.at[pl.ds(base, RPT)],
                    acc_out.at[c, pl.ds(base, RPT)])

  return seg_kernel(edge_idx, x)


def _tc_finish(p0, p1, deg, W, b2):
  M_BLK = 2048
  grid = (ACC_ROWS // M_BLK,)

  def body(p0_ref, p1_ref, d_ref, w_ref, b_ref, o_ref):
    seg = p0_ref[...] + p1_ref[...]
    degcol = jnp.sum(d_ref[...], axis=0)[:, None]
    degc = jnp.maximum(degcol, 1.0)
    y = lax.dot_general(seg / degc, w_ref[...], (((1,), (0,)), ((), ())),
                        precision=lax.Precision.HIGHEST,
                        preferred_element_type=jnp.float32)
    o_ref[...] = y + b_ref[...] * (degcol > 0.0).astype(jnp.float32)

  return pl.pallas_call(
      body,
      grid=grid,
      in_specs=[
          pl.BlockSpec((M_BLK, D), lambda i: (i, 0)),
          pl.BlockSpec((M_BLK, D), lambda i: (i, 0)),
          pl.BlockSpec((NW, M_BLK), lambda i: (0, i)),
          pl.BlockSpec((D, D), lambda i: (0, 0)),
          pl.BlockSpec((1, D), lambda i: (0, 0)),
      ],
      out_specs=pl.BlockSpec((M_BLK, D), lambda i: (i, 0)),
      out_shape=jax.ShapeDtypeStruct((ACC_ROWS, D), jnp.float32),
  )(p0, p1, deg, W, b2)


def kernel(x, edge_index, W, b):
  dst = edge_index[0]
  src = edge_index[1]
  pad = NW * EPW_PAD - N_EDGES
  src_p = jnp.concatenate([src, jnp.zeros((pad,), jnp.int32)])
  dst_p = jnp.concatenate(
      [dst, N_NODES + (jnp.arange(pad, dtype=jnp.int32) % (ACC_ROWS - N_NODES))])
  src_p = src_p.reshape(NW, CH, 1, CS)
  dst_p = dst_p.reshape(NW, CH, 1, CS)
  edge_idx = jnp.concatenate([src_p, dst_p], axis=2)

  acc, deg = _sc_segment_sum(edge_idx, x)
  out = _tc_finish(acc[0], acc[1], deg.reshape(NW, ACC_ROWS), W,
                   b.reshape(1, D))
  return out[:N_NODES]


# speedup vs baseline: 1.3970x; 1.3970x over previous
import functools

import jax
import jax.numpy as jnp
from jax import lax
from jax.experimental import pallas as pl
from jax.experimental.pallas import tpu as pltpu
from jax.experimental.pallas import tpu_sc as plsc

N_NODES = 10000
N_EDGES = 320000
D = 128

NC = 2
NS = 16
NW = NC * NS
CS = 128
EPW = N_EDGES // NW
CH = (EPW + CS - 1) // CS
EPW_PAD = CH * CS
ACC_ROWS = 10240
RPT = ACC_ROWS // NS
L = 16


def _sc_segment_sum(src_idx, dst_idx, x):
  mesh = plsc.VectorSubcoreMesh(core_axis_name="c", subcore_axis_name="s")

  @functools.partial(
      pl.kernel,
      mesh=mesh,
      out_type=[
          jax.ShapeDtypeStruct((NC, ACC_ROWS, D), jnp.float32),
          jax.ShapeDtypeStruct((NC, NS, ACC_ROWS), jnp.float32),
      ],
      scratch_types=[
          pltpu.VMEM((CH, CS), jnp.int32),
          pltpu.VMEM((CH, CS), jnp.int32),
          pltpu.VMEM((CS, D), jnp.float32),
          pltpu.VMEM((ACC_ROWS,), jnp.float32),
          pltpu.VMEM_SHARED((ACC_ROWS, D), jnp.float32),
          pltpu.SemaphoreType.DMA,
      ],
      compiler_params=pltpu.CompilerParams(needs_layout_passes=False),
  )
  def seg_kernel(src_hbm, dst_hbm, x_hbm, acc_out, deg_out,
                 src_v, dst_v, rows_v, deg_v, acc_sh, sem):
    c = lax.axis_index("c")
    s = lax.axis_index("s")
    wid = c * NS + s

    zl = jnp.zeros((L,), jnp.float32)

    def fill_zrows(r, _):
      for q in range(D // L):
        rows_v[r, pl.ds(q * L, L)] = zl
      return 0
    lax.fori_loop(0, CS, fill_zrows, 0)

    def fill_deg(i, _):
      for q in range(8):
        deg_v[pl.ds((8 * i + q) * L, L)] = zl
      return 0
    lax.fori_loop(0, ACC_ROWS // (8 * L), fill_deg, 0)

    base = s * RPT
    for k in range(RPT // CS):
      pltpu.sync_copy(rows_v, acc_sh.at[pl.ds(base + k * CS, CS)])
    plsc.subcore_barrier()

    pltpu.sync_copy(src_hbm.at[wid], src_v)
    pltpu.sync_copy(dst_hbm.at[wid], dst_v)

    ones_l = jnp.ones((L,), jnp.float32)

    def chunk(j, _):
      gather = pltpu.async_copy(x_hbm.at[src_v.at[j]], rows_v, sem)
      for g in range(CS // L):
        plsc.addupdate_scatter(deg_v, [dst_v[j, pl.ds(g * L, L)]], ones_l)
      gather.wait()
      pltpu.sync_copy(rows_v, acc_sh.at[dst_v.at[j]], add=True)
      return 0
    lax.fori_loop(0, CH, chunk, 0)

    pltpu.sync_copy(deg_v, deg_out.at[c, s])
    plsc.subcore_barrier()
    pltpu.sync_copy(acc_sh.at[pl.ds(base, RPT)],
                    acc_out.at[c, pl.ds(base, RPT)])

  return seg_kernel(src_idx, dst_idx, x)


def _tc_finish(p0, p1, deg, W, b2):
  M_BLK = 2048
  grid = (ACC_ROWS // M_BLK,)

  def body(p0_ref, p1_ref, d_ref, w_ref, b_ref, o_ref):
    seg = p0_ref[...] + p1_ref[...]
    degcol = jnp.sum(d_ref[...], axis=0)[:, None]
    degc = jnp.maximum(degcol, 1.0)
    y = lax.dot_general(seg / degc, w_ref[...], (((1,), (0,)), ((), ())),
                        precision=lax.Precision.HIGHEST,
                        preferred_element_type=jnp.float32)
    o_ref[...] = y + b_ref[...] * (degcol > 0.0).astype(jnp.float32)

  return pl.pallas_call(
      body,
      grid=grid,
      in_specs=[
          pl.BlockSpec((M_BLK, D), lambda i: (i, 0)),
          pl.BlockSpec((M_BLK, D), lambda i: (i, 0)),
          pl.BlockSpec((NW, M_BLK), lambda i: (0, i)),
          pl.BlockSpec((D, D), lambda i: (0, 0)),
          pl.BlockSpec((1, D), lambda i: (0, 0)),
      ],
      out_specs=pl.BlockSpec((M_BLK, D), lambda i: (i, 0)),
      out_shape=jax.ShapeDtypeStruct((ACC_ROWS, D), jnp.float32),
  )(p0, p1, deg, W, b2)


def kernel(x, edge_index, W, b):
  dst = edge_index[0]
  src = edge_index[1]
  pad = NW * EPW_PAD - N_EDGES
  src_p = jnp.concatenate([src, jnp.zeros((pad,), jnp.int32)])
  dst_p = jnp.concatenate(
      [dst, N_NODES + (jnp.arange(pad, dtype=jnp.int32) % (ACC_ROWS - N_NODES))])
  src_p = src_p.reshape(NW, CH, CS)
  dst_p = dst_p.reshape(NW, CH, CS)

  acc, deg = _sc_segment_sum(src_p, dst_p, x)
  out = _tc_finish(acc[0], acc[1], deg.reshape(NW, ACC_ROWS), W,
                   b.reshape(1, D))
  return out[:N_NODES]
